# Initial kernel scaffold; baseline (speedup 1.0000x reference)
import jax
import jax.numpy as jnp
from jax.experimental import pallas as pl


def kernel(x, edge_index, conv_weight):
    raise NotImplementedError("write your pallas kernel here")


# speedup vs baseline: 3.9088x; 3.9088x over previous
import functools
import math

import jax
import jax.numpy as jnp
from jax import lax
from jax.experimental import pallas as pl
from jax.experimental.pallas import tpu as pltpu
from jax.experimental.pallas import tpu_sc as plsc

N = 10000
D = 256
E = 160000
K = 5
FN = 5
NPAD = 10240
EPAD = 163840
ECHUNKS = 1280
CPW = ECHUNKS // 16
RPW = NPAD // 16


def _bern_coef(degree):
    out = []
    for k in range(degree + 1):
        row = [0.0] * (degree + 1)
        for j in range(degree - k + 1):
            row[k + j] = float(math.comb(degree, k) * math.comb(degree - k, j)
                               * (-1) ** j)
        out.append(row)
    return out


COEF = _bern_coef(K)


def _sc_body(xs, srcp, dstp, bx, curp,
             acc,
             sidx, didx, stage, abuf, cbuf, rsbuf, sem):
    c = lax.axis_index("c")
    s = lax.axis_index("s")
    lanes = jax.lax.iota(jnp.int32, 16)

    def _fillstage(val):
        def _f(i, _):
            for q in range(8):
                stage[i, pl.ds(q * 16, 16)] = jnp.full((16,), val, jnp.float32)
            return 0
        lax.fori_loop(0, 128, _f, 0, unroll=2)

    def _zacc(j, _):
        pltpu.sync_copy(stage, acc.at[pl.ds(s * RPW + j * 128, 128)])
        return 0

    _fillstage(0.0)
    lax.fori_loop(0, RPW // 128, _zacc, 0)
    plsc.subcore_barrier()
    _fillstage(1.0)

    def _deg(j, _):
        pltpu.sync_copy(srcp.at[s * CPW + j], sidx)
        pltpu.sync_copy(stage, acc.at[sidx], add=True)
        return 0
    lax.fori_loop(0, CPW, _deg, 0)
    plsc.subcore_barrier()

    def _rsch(k2, _):
        pltpu.sync_copy(acc.at[pl.ds(s * RPW + k2 * 64, 64)], abuf)

        def _rs(g, _):
            rvec = jnp.zeros((16,), jnp.float32)
            for jj in range(16):
                d = abuf[g * 16 + jj, pl.ds(0, 16)]
                y = jnp.full((16,), 1.0, jnp.float32)
                for _ in range(10):
                    y = jnp.where(d * y * y > 2.0, 0.5 * y, y)
                for _ in range(6):
                    y = y * (1.5 - 0.5 * d * y * y)
                y = jnp.where(d >= 0.5, y, 0.0)
                rvec = jnp.where(lanes == jj, y, rvec)
            rsbuf[pl.ds(k2 * 64 + g * 16, 16)] = rvec
            return 0
        lax.fori_loop(0, 4, _rs, 0)
        return 0
    lax.fori_loop(0, RPW // 64, _rsch, 0)

    _fillstage(0.0)
    lax.fori_loop(0, RPW // 128, _zacc, 0)

    def _initp(k2, _):
        r = s * RPW + k2 * 64
        pltpu.sync_copy(xs.at[c, pl.ds(r, 64)], cbuf)

        def _grp(g, _):
            rv16 = rsbuf[pl.ds(k2 * 64 + g * 16, 16)]
            for jj in range(16):
                rv = rv16[jj]
                i = g * 16 + jj
                for q in range(8):
                    cbuf[i, pl.ds(q * 16, 16)] = (
                        rv * cbuf[i, pl.ds(q * 16, 16)])
            return 0
        lax.fori_loop(0, 4, _grp, 0)
        pltpu.sync_copy(cbuf, curp.at[c, pl.ds(r, 64)])
        return 0
    lax.fori_loop(0, RPW // 64, _initp, 0)
    plsc.subcore_barrier()

    def _hop(t, _):
        def _edge(j, _):
            pltpu.sync_copy(srcp.at[s * CPW + j], sidx)
            pltpu.async_copy(curp.at[c].at[sidx], stage, sem).wait()
            pltpu.sync_copy(dstp.at[s * CPW + j], didx)
            pltpu.sync_copy(stage, acc.at[didx], add=True)
            return 0
        lax.fori_loop(0, CPW, _edge, 0)
        plsc.subcore_barrier()

        _fillstage(0.0)

        def _wb(k2, _):
            r = s * RPW + k2 * 64
            pltpu.sync_copy(acc.at[pl.ds(r, 64)], abuf)
            pltpu.sync_copy(stage.at[pl.ds(0, 64)], acc.at[pl.ds(r, 64)])

            @pl.when(t == 0)
            def _():
                pltpu.sync_copy(xs.at[c, pl.ds(r, 64)], cbuf)

            @pl.when(t > 0)
            def _():
                pltpu.sync_copy(bx.at[t - 1, c, pl.ds(r, 64)], cbuf)

            def _grp(g, _):
                rv16 = rsbuf[pl.ds(k2 * 64 + g * 16, 16)]
                for jj in range(16):
                    rv = rv16[jj]
                    i = g * 16 + jj
                    for q in range(8):
                        av = abuf[i, pl.ds(q * 16, 16)]
                        cv = cbuf[i, pl.ds(q * 16, 16)]
                        nv = cv - rv * av
                        abuf[i, pl.ds(q * 16, 16)] = nv
                        cbuf[i, pl.ds(q * 16, 16)] = rv * nv
                return 0
            lax.fori_loop(0, 4, _grp, 0)
            pltpu.sync_copy(abuf, bx.at[t, c, pl.ds(r, 64)])
            pltpu.sync_copy(cbuf, curp.at[c, pl.ds(r, 64)])
            return 0
        lax.fori_loop(0, RPW // 64, _wb, 0)
        plsc.subcore_barrier()
        return 0
    lax.fori_loop(0, K, _hop, 0)


def _sc_propagate(xs, srcp, dstp):
    mesh = plsc.VectorSubcoreMesh(core_axis_name="c", subcore_axis_name="s",
                                  num_cores=2, num_subcores=16)
    f = pl.kernel(
        _sc_body,
        out_type=[
            jax.ShapeDtypeStruct((K, 2, NPAD, 128), jnp.float32),
            jax.ShapeDtypeStruct((2, NPAD, 128), jnp.float32),
        ],
        mesh=mesh,
        scratch_types=[
            pltpu.VMEM_SHARED((NPAD, 128), jnp.float32),
            pltpu.VMEM((128,), jnp.int32),
            pltpu.VMEM((128,), jnp.int32),
            pltpu.VMEM((128, 128), jnp.float32),
            pltpu.VMEM((64, 128), jnp.float32),
            pltpu.VMEM((64, 128), jnp.float32),
            pltpu.VMEM((RPW,), jnp.float32),
            pltpu.SemaphoreType.DMA,
        ],
    )
    bxs, _ = f(xs, srcp, dstp)
    return bxs


def _combine_body(x_ref, bx_ref, out_ref):
    for h in range(2):
        xh = x_ref[:, h * 128:(h + 1) * 128]
        bs = [bx_ref[i, h] for i in range(K)]
        for k in range(K + 1):
            ck = COEF[k]
            accv = xh * ck[0] if ck[0] != 0.0 else jnp.zeros_like(xh)
            for i in range(1, K + 1):
                if ck[i] != 0.0:
                    accv = accv + bs[i - 1] * ck[i]
            for f in range(FN):
                out_ref[:, f, pl.ds(k * 256 + h * 128, 128)] = accv


def _tc_combine(x, bxs):
    bn = 200
    grid = (N // bn,)
    return pl.pallas_call(
        _combine_body,
        grid=grid,
        in_specs=[
            pl.BlockSpec((bn, D), lambda n: (n, 0)),
            pl.BlockSpec((K, 2, bn, 128), lambda n: (0, 0, n, 0)),
        ],
        out_specs=pl.BlockSpec((bn, FN, (K + 1) * D), lambda n: (n, 0, 0)),
        out_shape=jax.ShapeDtypeStruct((N, FN, (K + 1) * D), jnp.float32),
    )(x, bxs)


def kernel(x, edge_index, conv_weight):
    del conv_weight
    x = x.astype(jnp.float32)
    ei = edge_index.astype(jnp.int32)
    pad = jnp.full((EPAD - E,), N, jnp.int32)
    srcp = jnp.concatenate([ei[:, 0], pad]).reshape(ECHUNKS, 128)
    dstp = jnp.concatenate([ei[:, 1], pad]).reshape(ECHUNKS, 128)
    xp = jnp.concatenate([x, jnp.zeros((NPAD - N, D), jnp.float32)])
    xs = jnp.stack([xp[:, :128], xp[:, 128:]])
    bxs = _sc_propagate(xs, srcp, dstp)
    return _tc_combine(x, bxs)


# speedup vs baseline: 5.2578x; 1.3451x over previous
import functools
import math

import jax
import jax.numpy as jnp
from jax import lax
from jax.experimental import pallas as pl
from jax.experimental.pallas import tpu as pltpu
from jax.experimental.pallas import tpu_sc as plsc

N = 10000
D = 256
E = 160000
K = 5
FN = 5
NPAD = 10240
EPAD = 163840
ECHUNKS = 1280
CPW = ECHUNKS // 16
RPW = NPAD // 16


def _bern_coef(degree):
    out = []
    for k in range(degree + 1):
        row = [0.0] * (degree + 1)
        for j in range(degree - k + 1):
            row[k + j] = float(math.comb(degree, k) * math.comb(degree - k, j)
                               * (-1) ** j)
        out.append(row)
    return out


COEF = _bern_coef(K)


def _sc_body(xs, sdp, bx, curp,
             acc,
             sd, stage0, stage1, rsbuf, gsem0, gsem1):
    c = lax.axis_index("c")
    s = lax.axis_index("s")
    lanes = jax.lax.iota(jnp.int32, 16)
    base = s * CPW

    def _fill(ref, nrows, val):
        def _f(i, _):
            for q in range(8):
                ref[i, pl.ds(q * 16, 16)] = jnp.full((16,), val, jnp.float32)
            return 0
        lax.fori_loop(0, nrows, _f, 0, unroll=2)

    def _zacc(j, _):
        pltpu.sync_copy(stage0, acc.at[pl.ds(s * RPW + j * 128, 128)])
        return 0

    _fill(stage0, 128, 0.0)
    lax.fori_loop(0, RPW // 128, _zacc, 0)
    plsc.subcore_barrier()
    _fill(stage1, 128, 1.0)

    def _deg(j, _):
        pltpu.sync_copy(sdp.at[base + j], sd.at[pl.ds(0, 2)])
        pltpu.sync_copy(stage1, acc.at[sd.at[0]], add=True)
        return 0
    lax.fori_loop(0, CPW, _deg, 0)
    plsc.subcore_barrier()

    def _rsch(k2, _):
        pltpu.sync_copy(acc.at[pl.ds(s * RPW + k2 * 64, 64)], stage1.at[pl.ds(0, 64)])

        def _rs(g, _):
            rvec = jnp.zeros((16,), jnp.float32)
            for jj in range(16):
                d = stage1[g * 16 + jj, pl.ds(0, 16)]
                y = jnp.full((16,), 1.0, jnp.float32)
                for _ in range(10):
                    y = jnp.where(d * y * y > 2.0, 0.5 * y, y)
                for _ in range(6):
                    y = y * (1.5 - 0.5 * d * y * y)
                y = jnp.where(d >= 0.5, y, 0.0)
                rvec = jnp.where(lanes == jj, y, rvec)
            rsbuf[pl.ds(k2 * 64 + g * 16, 16)] = rvec
            return 0
        lax.fori_loop(0, 4, _rs, 0)
        return 0
    lax.fori_loop(0, RPW // 64, _rsch, 0)

    lax.fori_loop(0, RPW // 128, _zacc, 0)

    def _initp(k2, _):
        r = s * RPW + k2 * 64
        pltpu.sync_copy(xs.at[c, pl.ds(r, 64)], stage1.at[pl.ds(0, 64)])

        def _grp(g, _):
            rv16 = rsbuf[pl.ds(k2 * 64 + g * 16, 16)]
            for jj in range(16):
                rv = rv16[jj]
                i = g * 16 + jj
                for q in range(8):
                    stage1[i, pl.ds(q * 16, 16)] = (
                        rv * stage1[i, pl.ds(q * 16, 16)])
            return 0
        lax.fori_loop(0, 4, _grp, 0)
        pltpu.sync_copy(stage1.at[pl.ds(0, 64)], curp.at[c, pl.ds(r, 64)])
        return 0
    lax.fori_loop(0, RPW // 64, _initp, 0)
    plsc.subcore_barrier()

    def _hop(t, _):
        pltpu.sync_copy(sdp.at[base], sd.at[pl.ds(0, 2)])
        pltpu.async_copy(curp.at[c].at[sd.at[0]], stage0, gsem0)

        def _pair(jj, _):
            cb = base + 2 * jj + 1
            pltpu.sync_copy(sdp.at[cb], sd.at[pl.ds(2, 2)])
            pltpu.async_copy(curp.at[c].at[sd.at[2]], stage1, gsem1)
            pltpu.make_async_copy(curp.at[c].at[sd.at[0]], stage0,
                                  gsem0).wait()
            pltpu.sync_copy(stage0, acc.at[sd.at[1]], add=True)
            cc = jnp.minimum(cb + 1, base + CPW - 1)
            pltpu.sync_copy(sdp.at[cc], sd.at[pl.ds(0, 2)])
            pltpu.async_copy(curp.at[c].at[sd.at[0]], stage0, gsem0)
            pltpu.make_async_copy(curp.at[c].at[sd.at[2]], stage1,
                                  gsem1).wait()
            pltpu.sync_copy(stage1, acc.at[sd.at[3]], add=True)
            return 0
        lax.fori_loop(0, CPW // 2, _pair, 0)
        pltpu.make_async_copy(curp.at[c].at[sd.at[0]], stage0, gsem0).wait()
        plsc.subcore_barrier()

        _fill(stage0, 64, 0.0)

        def _wb(k2, _):
            r = s * RPW + k2 * 64
            pltpu.sync_copy(acc.at[pl.ds(r, 64)], stage0.at[pl.ds(64, 64)])
            pltpu.sync_copy(stage0.at[pl.ds(0, 64)], acc.at[pl.ds(r, 64)])

            @pl.when(t == 0)
            def _():
                pltpu.sync_copy(xs.at[c, pl.ds(r, 64)], stage1.at[pl.ds(0, 64)])

            @pl.when(t > 0)
            def _():
                pltpu.sync_copy(bx.at[t - 1, c, pl.ds(r, 64)],
                                stage1.at[pl.ds(0, 64)])

            def _grp(g, _):
                rv16 = rsbuf[pl.ds(k2 * 64 + g * 16, 16)]
                for jj in range(16):
                    rv = rv16[jj]
                    i = g * 16 + jj
                    for q in range(8):
                        av = stage0[64 + i, pl.ds(q * 16, 16)]
                        cv = stage1[i, pl.ds(q * 16, 16)]
                        nv = cv - rv * av
                        stage0[64 + i, pl.ds(q * 16, 16)] = nv
                        stage1[i, pl.ds(q * 16, 16)] = rv * nv
                return 0
            lax.fori_loop(0, 4, _grp, 0)
            pltpu.sync_copy(stage0.at[pl.ds(64, 64)], bx.at[t, c, pl.ds(r, 64)])
            pltpu.sync_copy(stage1.at[pl.ds(0, 64)], curp.at[c, pl.ds(r, 64)])
            return 0
        lax.fori_loop(0, RPW // 64, _wb, 0)
        plsc.subcore_barrier()
        return 0
    lax.fori_loop(0, K, _hop, 0)


def _sc_propagate(xs, sdp):
    mesh = plsc.VectorSubcoreMesh(core_axis_name="c", subcore_axis_name="s",
                                  num_cores=2, num_subcores=16)
    f = pl.kernel(
        _sc_body,
        out_type=[
            jax.ShapeDtypeStruct((K, 2, NPAD, 128), jnp.float32),
            jax.ShapeDtypeStruct((2, NPAD, 128), jnp.float32),
        ],
        mesh=mesh,
        scratch_types=[
            pltpu.VMEM_SHARED((NPAD, 128), jnp.float32),
            pltpu.VMEM((4, 128), jnp.int32),
            pltpu.VMEM((128, 128), jnp.float32),
            pltpu.VMEM((128, 128), jnp.float32),
            pltpu.VMEM((RPW,), jnp.float32),
            pltpu.SemaphoreType.DMA,
            pltpu.SemaphoreType.DMA,
        ],
    )
    bxs, _ = f(xs, sdp)
    return bxs


def _combine_body(x_ref, bx_ref, out_ref):
    for h in range(2):
        xh = x_ref[:, h * 128:(h + 1) * 128]
        bs = [bx_ref[i, h] for i in range(K)]
        for k in range(K + 1):
            ck = COEF[k]
            accv = xh * ck[0] if ck[0] != 0.0 else jnp.zeros_like(xh)
            for i in range(1, K + 1):
                if ck[i] != 0.0:
                    accv = accv + bs[i - 1] * ck[i]
            for f in range(FN):
                out_ref[:, f, pl.ds(k * 256 + h * 128, 128)] = accv


def _tc_combine(x, bxs):
    bn = 200
    grid = (N // bn,)
    return pl.pallas_call(
        _combine_body,
        grid=grid,
        in_specs=[
            pl.BlockSpec((bn, D), lambda n: (n, 0)),
            pl.BlockSpec((K, 2, bn, 128), lambda n: (0, 0, n, 0)),
        ],
        out_specs=pl.BlockSpec((bn, FN, (K + 1) * D), lambda n: (n, 0, 0)),
        out_shape=jax.ShapeDtypeStruct((N, FN, (K + 1) * D), jnp.float32),
    )(x, bxs)


def kernel(x, edge_index, conv_weight):
    del conv_weight
    x = x.astype(jnp.float32)
    ei = edge_index.astype(jnp.int32)
    pad = jnp.full((EPAD - E,), N, jnp.int32)
    srcp = jnp.concatenate([ei[:, 0], pad]).reshape(ECHUNKS, 128)
    dstp = jnp.concatenate([ei[:, 1], pad]).reshape(ECHUNKS, 128)
    sdp = jnp.stack([srcp, dstp], axis=1)
    xp = jnp.concatenate([x, jnp.zeros((NPAD - N, D), jnp.float32)])
    xs = jnp.stack([xp[:, :128], xp[:, 128:]])
    bxs = _sc_propagate(xs, sdp)
    return _tc_combine(x, bxs)
